# baseline (speedup 1.0000x reference)
import functools

import jax
import jax.numpy as jnp
from jax import lax
from jax.experimental import pallas as pl
from jax.experimental.pallas import tpu as pltpu
from jax.experimental.pallas import tpu_sc as plsc

B, R, D, K = 4096, 100, 128, 64
NW = 32
BW = B // NW
G2 = BW // 2


def _sc_gather(tensor, indices):
    mesh = plsc.VectorSubcoreMesh(core_axis_name="c", subcore_axis_name="s")

    @functools.partial(
        pl.kernel,
        mesh=mesh,
        out_type=jax.ShapeDtypeStruct((B, R, K), jnp.float32),
        scratch_types=[
            pltpu.VMEM((R, D), jnp.float32),
            pltpu.VMEM((R, D), jnp.float32),
            pltpu.VMEM((R, K), jnp.int32),
            pltpu.VMEM((R, K), jnp.int32),
            pltpu.VMEM((R, K), jnp.float32),
            pltpu.VMEM((R, K), jnp.float32),
            pltpu.SemaphoreType.DMA,
            pltpu.SemaphoreType.DMA,
            pltpu.SemaphoreType.DMA,
            pltpu.SemaphoreType.DMA,
        ],
        compiler_params=pltpu.CompilerParams(
            needs_layout_passes=False, use_tc_tiling_on_sc=True),
    )
    def k(t_hbm, i_hbm, o_hbm, rows0, rows1, idx0, idx1, out0, out1,
          si0, si1, so0, so1):
        wid = lax.axis_index("s") * 2 + lax.axis_index("c")
        b0 = wid * BW
        rows, idxv, outv = (rows0, rows1), (idx0, idx1), (out0, out1)
        sin, sout = (si0, si1), (so0, so1)

        def start_load(ci, b):
            pltpu.make_async_copy(t_hbm.at[b0 + ci], rows[b], sin[b]).start()
            pltpu.make_async_copy(i_hbm.at[b0 + ci], idxv[b], sin[b]).start()

        def wait_load(b):
            pltpu.make_async_copy(t_hbm.at[b0], rows[b], sin[b]).wait()
            pltpu.make_async_copy(i_hbm.at[b0], idxv[b], sin[b]).wait()

        def start_store(ci, b):
            pltpu.make_async_copy(outv[b], o_hbm.at[b0 + ci], sout[b]).start()

        def wait_store(b):
            pltpu.make_async_copy(outv[b], o_hbm.at[b0], sout[b]).wait()

        def compute(b):
            @plsc.parallel_loop(0, R, 1, unroll=4)
            def row_body(r):
                rvec = jnp.full((16,), r, jnp.int32)
                for j in range(K // 16):
                    col = idxv[b][r, pl.ds(j * 16, 16)]
                    outv[b][r, pl.ds(j * 16, 16)] = plsc.load_gather(
                        rows[b], [rvec, col])

        start_load(0, 0)
        start_load(1, 1)
        for b in (0, 1):
            wait_load(b)
            compute(b)
            start_store(b, b)
            start_load(b + 2, b)

        def body(g, carry):
            for b in (0, 1):
                ci = 2 * g + b
                wait_load(b)
                wait_store(b)
                compute(b)
                start_store(ci, b)
                start_load(ci + 2, b)
            return carry

        lax.fori_loop(1, G2 - 1, body, 0)

        for b in (0, 1):
            wait_load(b)
            wait_store(b)
            compute(b)
            start_store(2 * (G2 - 1) + b, b)
        for b in (0, 1):
            wait_store(b)

    return k(tensor, indices)


def kernel(tensor, indices):
    return _sc_gather(tensor, indices)


# speedup vs baseline: 4.1346x; 4.1346x over previous
import functools

import jax
import jax.numpy as jnp
from jax import lax
from jax.experimental import pallas as pl
from jax.experimental.pallas import tpu as pltpu
from jax.experimental.pallas import tpu_sc as plsc

B, R, D, K = 4096, 100, 128, 64
NW = 32
BW = B // NW
CR = 2
NCHUNK = R // CR
G2 = NCHUNK // 2


def _sc_gather(t2, i2):
    mesh = plsc.VectorSubcoreMesh(core_axis_name="c", subcore_axis_name="s")

    @functools.partial(
        pl.kernel,
        mesh=mesh,
        out_type=jax.ShapeDtypeStruct((R, K, B), jnp.float32),
        scratch_types=[
            pltpu.VMEM((CR, BW, D), jnp.float32),
            pltpu.VMEM((CR, BW, D), jnp.float32),
            pltpu.VMEM((CR, K, BW), jnp.int32),
            pltpu.VMEM((CR, K, BW), jnp.int32),
            pltpu.VMEM((K, BW), jnp.float32),
            pltpu.VMEM((K, BW), jnp.float32),
            pltpu.SemaphoreType.DMA,
            pltpu.SemaphoreType.DMA,
            pltpu.SemaphoreType.DMA,
            pltpu.SemaphoreType.DMA,
        ],
        compiler_params=pltpu.CompilerParams(
            needs_layout_passes=False, use_tc_tiling_on_sc=True),
    )
    def k(t_hbm, i_hbm, o_hbm, rows0, rows1, idx0, idx1, out0, out1,
          si0, si1, so0, so1):
        wid = lax.axis_index("s") * 2 + lax.axis_index("c")
        b0 = wid * BW
        rows, idxv, outv = (rows0, rows1), (idx0, idx1), (out0, out1)
        sin, sout = (si0, si1), (so0, so1)

        def start_load(ci, b):
            pltpu.make_async_copy(
                t_hbm.at[pl.ds(ci * CR, CR), pl.ds(b0, BW)],
                rows[b], sin[b]).start()
            pltpu.make_async_copy(
                i_hbm.at[pl.ds(ci * CR, CR), :, pl.ds(b0, BW)],
                idxv[b], sin[b]).start()

        def wait_load(b):
            pltpu.make_async_copy(
                t_hbm.at[pl.ds(0, CR), pl.ds(b0, BW)], rows[b], sin[b]).wait()
            pltpu.make_async_copy(
                i_hbm.at[pl.ds(0, CR), :, pl.ds(b0, BW)],
                idxv[b], sin[b]).wait()

        def start_store(ci, rr):
            pltpu.make_async_copy(
                outv[rr], o_hbm.at[ci * CR + rr, :, pl.ds(b0, BW)],
                sout[rr]).start()

        def wait_store(rr):
            pltpu.make_async_copy(
                outv[rr], o_hbm.at[0, :, pl.ds(b0, BW)], sout[rr]).wait()

        lanes = lax.iota(jnp.int32, 16)
        bvecs = [lanes + (g * 16) for g in range(BW // 16)]

        def compute(b, rr):
            rvec = jnp.full((16,), rr, jnp.int32)

            @plsc.parallel_loop(0, K, 1, unroll=4)
            def j_body(j):
                for g in range(BW // 16):
                    col = idxv[b][rr, j, pl.ds(g * 16, 16)]
                    outv[rr][j, pl.ds(g * 16, 16)] = plsc.load_gather(
                        rows[b], [rvec, bvecs[g], col])

        def do_chunk(ci, b, first):
            wait_load(b)
            for rr in range(CR):
                if not first:
                    wait_store(rr)
                compute(b, rr)
                start_store(ci, rr)

        start_load(0, 0)
        start_load(1, 1)
        for b in (0, 1):
            do_chunk(b, b, first=(b == 0))
            start_load(b + 2, b)

        def body(g, carry):
            for b in (0, 1):
                ci = 2 * g + b
                do_chunk(ci, b, first=False)
                start_load(ci + 2, b)
            return carry

        lax.fori_loop(1, G2 - 1, body, 0)

        for b in (0, 1):
            do_chunk(2 * (G2 - 1) + b, b, first=False)
        for rr in range(CR):
            wait_store(rr)

    return k(t2, i2)


def kernel(tensor, indices):
    t2 = jnp.transpose(tensor, (1, 0, 2))
    i2 = jnp.transpose(indices, (1, 2, 0))
    out2 = _sc_gather(t2, i2)
    return jnp.transpose(out2, (2, 0, 1))
